# baseline (speedup 1.0000x reference)
import functools

import jax
import jax.numpy as jnp
from jax import lax
from jax.experimental import pallas as pl
from jax.experimental.pallas import tpu as pltpu
from jax.experimental.pallas import tpu_sc as plsc

N = 10000
E = 320000
HID = 128
NC = 2
NS = 16
NW = NC * NS
CK = 80
NCH = 128
EPT = CK * NCH
EP = EPT * NW
NP = 10016
RPT = NP // NS
RB = 1000

_mesh = functools.partial(
    plsc.VectorSubcoreMesh, core_axis_name="c", subcore_axis_name="s"
)


def _seg_sum(table, packed, zeros, d, split):
    def unpack(packed_v, j, scv, dcv):
        for v in range(CK // 16):
            p = packed_v[j, pl.ds(v * 16, 16)]
            scv[pl.ds(v * 16, 16)] = lax.bitwise_and(p, jnp.int32(0xFFFF))
            dcv[pl.ds(v * 16, 16)] = lax.shift_right_logical(p, 16)

    t0, t1 = split
    tmax = min(NCH, max(t0, t1))

    @functools.partial(
        pl.kernel,
        out_type=jax.ShapeDtypeStruct((NC, NP, d), jnp.float32),
        mesh=_mesh(),
        compiler_params=pltpu.CompilerParams(use_tc_tiling_on_sc=False),
        scratch_types=[
            pltpu.VMEM((tmax, CK), jnp.int32),
            pltpu.VMEM((CK,), jnp.int32),
            pltpu.VMEM((CK,), jnp.int32),
            pltpu.VMEM((CK,), jnp.int32),
            pltpu.VMEM((CK,), jnp.int32),
            pltpu.VMEM((CK, d), jnp.float32),
            pltpu.VMEM((CK, d), jnp.float32),
            pltpu.VMEM_SHARED((NP, d), jnp.float32),
            pltpu.SemaphoreType.DMA,
            pltpu.SemaphoreType.DMA,
        ],
    )
    def kern(table_h, packed_h, zeros_h, out_h, packed_v, sca, dca, scb,
             dcb, stage_a, stage_b, acc_s, sem_a, sem_b):
        cid = lax.axis_index("c")
        sid = lax.axis_index("s")
        row0 = sid * RPT
        pltpu.sync_copy(zeros_h, acc_s.at[pl.ds(row0, RPT)])

        def run(nch, gbase):
            pltpu.sync_copy(packed_h.at[pl.ds(gbase, nch)],
                            packed_v.at[pl.ds(0, nch)])
            plsc.subcore_barrier()
            unpack(packed_v, 0, sca, dca)
            pltpu.async_copy(table_h.at[sca], stage_a, sem_a)

            def body(t, carry):
                j0 = 2 * t
                unpack(packed_v, j0 + 1, scb, dcb)
                pltpu.async_copy(table_h.at[scb], stage_b, sem_b)
                pltpu.make_async_copy(table_h.at[sca], stage_a,
                                      sem_a).wait()
                pltpu.sync_copy(stage_a, acc_s.at[dca], add=True)

                @pl.when(t < nch // 2 - 1)
                def _prefetch():
                    unpack(packed_v, j0 + 2, sca, dca)
                    pltpu.async_copy(table_h.at[sca], stage_a, sem_a)

                pltpu.make_async_copy(table_h.at[scb], stage_b,
                                      sem_b).wait()
                pltpu.sync_copy(stage_b, acc_s.at[dcb], add=True)
                return carry

            lax.fori_loop(0, nch // 2, body, 0)

        if t0 > 0:
            @pl.when(cid == 0)
            def _core0():
                for r0 in range(0, t0, tmax):
                    run(min(tmax, t0 - r0), sid * t0 + r0)

        if t1 > 0:
            @pl.when(cid == 1)
            def _core1():
                for r0 in range(0, t1, tmax):
                    run(min(tmax, t1 - r0), NS * t0 + sid * t1 + r0)

        plsc.subcore_barrier()
        pltpu.sync_copy(acc_s.at[pl.ds(row0, RPT)],
                        out_h.at[cid, pl.ds(row0, RPT)])

    return kern(table, packed, zeros)


def _deg_pass(dstp, ones, zeros):
    @functools.partial(
        pl.kernel,
        out_type=jax.ShapeDtypeStruct((NC, NP, 16), jnp.float32),
        mesh=_mesh(),
        compiler_params=pltpu.CompilerParams(use_tc_tiling_on_sc=False),
        scratch_types=[
            pltpu.VMEM((NCH, CK), jnp.int32),
            pltpu.VMEM((CK, 16), jnp.float32),
            pltpu.VMEM_SHARED((NP, 16), jnp.float32),
        ],
    )
    def kern(dst_h, ones_h, zeros_h, out_h, dst_v, stage_v, acc_s):
        cid = lax.axis_index("c")
        sid = lax.axis_index("s")
        wid = sid * NC + cid
        row0 = sid * RPT
        pltpu.sync_copy(ones_h, stage_v)
        pltpu.sync_copy(dst_h.at[pl.ds(wid * NCH, NCH)], dst_v)
        pltpu.sync_copy(zeros_h, acc_s.at[pl.ds(row0, RPT)])
        plsc.subcore_barrier()

        def chunk(j, carry):
            pltpu.sync_copy(stage_v, acc_s.at[dst_v.at[j]], add=True)
            return carry

        lax.fori_loop(0, NCH, chunk, 0)
        plsc.subcore_barrier()
        pltpu.sync_copy(acc_s.at[pl.ds(row0, RPT)],
                        out_h.at[cid, pl.ds(row0, RPT)])

    return kern(dstp, ones, zeros)


def _tc1(degp, x16):
    def body(degp_ref, x16_ref, dinv_ref, xs_ref):
        deg = degp_ref[0, 0:N, 0:1] + degp_ref[1, 0:N, 0:1] + 1.0
        dinv = lax.rsqrt(deg)
        dinv_ref[...] = dinv
        xs_ref[...] = dinv * x16_ref[...]

    return pl.pallas_call(
        body,
        out_shape=(
            jax.ShapeDtypeStruct((N, 1), jnp.float32),
            jax.ShapeDtypeStruct((N, 16), jnp.float32),
        ),
    )(degp, x16)


def _tc2(a1p, xs, dinv, w116, w2, b1):
    def body(a1p_ref, xs_ref, dinv_ref, w1_ref, w2_ref, b1_ref,
             h1_ref, g2_ref):
        dinv = dinv_ref[...]
        a = a1p_ref[0] + a1p_ref[1] + xs_ref[...]
        t = jnp.dot(a, w1_ref[...], preferred_element_type=jnp.float32)
        h1 = jnp.maximum(dinv * t + b1_ref[...], 0.0)
        h1_ref[...] = h1
        g2_ref[...] = dinv * jnp.dot(h1, w2_ref[...],
                                     preferred_element_type=jnp.float32)

    return pl.pallas_call(
        body,
        grid=(N // RB,),
        in_specs=[
            pl.BlockSpec((NC, RB, 16), lambda i: (0, i, 0)),
            pl.BlockSpec((RB, 16), lambda i: (i, 0)),
            pl.BlockSpec((RB, 1), lambda i: (i, 0)),
            pl.BlockSpec((16, HID), lambda i: (0, 0)),
            pl.BlockSpec((HID, HID), lambda i: (0, 0)),
            pl.BlockSpec((1, HID), lambda i: (0, 0)),
        ],
        out_specs=[
            pl.BlockSpec((RB, HID), lambda i: (i, 0)),
            pl.BlockSpec((RB, HID), lambda i: (i, 0)),
        ],
        out_shape=(
            jax.ShapeDtypeStruct((N, HID), jnp.float32),
            jax.ShapeDtypeStruct((N, HID), jnp.float32),
        ),
    )(a1p, xs, dinv, w116, w2, b1)


def _tc3(a2p, g2, dinv, b2):
    def body(a2p_ref, g2_ref, dinv_ref, b2_ref, out_ref):
        s = a2p_ref[0] + a2p_ref[1] + g2_ref[...]
        out_ref[...] = jnp.maximum(dinv_ref[...] * s + b2_ref[...], 0.0)

    return pl.pallas_call(
        body,
        grid=(N // RB,),
        in_specs=[
            pl.BlockSpec((NC, RB, HID), lambda i: (0, i, 0)),
            pl.BlockSpec((RB, HID), lambda i: (i, 0)),
            pl.BlockSpec((RB, 1), lambda i: (i, 0)),
            pl.BlockSpec((1, HID), lambda i: (0, 0)),
        ],
        out_specs=pl.BlockSpec((RB, HID), lambda i: (i, 0)),
        out_shape=jax.ShapeDtypeStruct((N, HID), jnp.float32),
    )(a2p, g2, dinv, b2)


def kernel(x, edge_index, W1, b1, W2, b2):
    src = edge_index[0].astype(jnp.int32)
    dst = edge_index[1].astype(jnp.int32)
    dstp = jnp.concatenate(
        [dst, jnp.full((EP - E,), N, jnp.int32)]).reshape(NW * NCH, CK)
    packed = jnp.concatenate(
        [src, jnp.zeros((EP - E,), jnp.int32)]).reshape(NW * NCH, CK) | (
            dstp << 16)
    x16 = jnp.pad(x, ((0, 0), (0, 16 - x.shape[1])))
    w116 = jnp.pad(W1, ((0, 16 - W1.shape[0]), (0, 0)))
    ones16 = jnp.ones((CK, 16), jnp.float32)
    zeros16 = jnp.zeros((RPT, 16), jnp.float32)
    zeros128 = jnp.zeros((RPT, HID), jnp.float32)

    degp = _deg_pass(dstp, ones16, zeros16)
    dinv, xs = _tc1(degp, x16)
    a1p = _seg_sum(xs, packed, zeros16, 16, (NCH, NCH))
    h1, g2 = _tc2(a1p, xs, dinv, w116, W2, b1.reshape(1, HID))
    a2p = _seg_sum(g2, packed, zeros128, HID, (2 * NCH, 0))
    return _tc3(a2p, g2, dinv, b2.reshape(1, HID))


# speedup vs baseline: 1.3040x; 1.3040x over previous
import functools

import jax
import jax.numpy as jnp
from jax import lax
from jax.experimental import pallas as pl
from jax.experimental.pallas import tpu as pltpu
from jax.experimental.pallas import tpu_sc as plsc

N = 10000
E = 320000
HID = 128
NC = 2
NS = 16
NW = NC * NS
CK = 128
NCH = 80
EPT = CK * NCH
EP = EPT * NW
NP = 10016
RPT = NP // NS
RB = 1000

_mesh = functools.partial(
    plsc.VectorSubcoreMesh, core_axis_name="c", subcore_axis_name="s"
)


def _seg_sum(table, packed, zeros, d, split):
    def unpack(packed_v, j, scv, dcv):
        for v in range(CK // 16):
            p = packed_v[j, pl.ds(v * 16, 16)]
            scv[pl.ds(v * 16, 16)] = lax.bitwise_and(p, jnp.int32(0xFFFF))
            dcv[pl.ds(v * 16, 16)] = lax.shift_right_logical(p, 16)

    t0, t1 = split
    tmax = min(NCH, max(t0, t1))

    @functools.partial(
        pl.kernel,
        out_type=jax.ShapeDtypeStruct((NC, NP, d), jnp.float32),
        mesh=_mesh(),
        compiler_params=pltpu.CompilerParams(use_tc_tiling_on_sc=False),
        scratch_types=[
            pltpu.VMEM((tmax, CK), jnp.int32),
            pltpu.VMEM((CK,), jnp.int32),
            pltpu.VMEM((CK,), jnp.int32),
            pltpu.VMEM((CK,), jnp.int32),
            pltpu.VMEM((CK,), jnp.int32),
            pltpu.VMEM((CK, d), jnp.float32),
            pltpu.VMEM((CK, d), jnp.float32),
            pltpu.VMEM_SHARED((NP, d), jnp.float32),
            pltpu.SemaphoreType.DMA,
            pltpu.SemaphoreType.DMA,
        ],
    )
    def kern(table_h, packed_h, zeros_h, out_h, packed_v, sca, dca, scb,
             dcb, stage_a, stage_b, acc_s, sem_a, sem_b):
        cid = lax.axis_index("c")
        sid = lax.axis_index("s")
        row0 = sid * RPT
        pltpu.sync_copy(zeros_h, acc_s.at[pl.ds(row0, RPT)])

        def run(nch, gbase):
            pltpu.sync_copy(packed_h.at[pl.ds(gbase, nch)],
                            packed_v.at[pl.ds(0, nch)])
            plsc.subcore_barrier()
            unpack(packed_v, 0, sca, dca)
            pltpu.async_copy(table_h.at[sca], stage_a, sem_a)

            def body(t, carry):
                j0 = 2 * t
                unpack(packed_v, j0 + 1, scb, dcb)
                pltpu.async_copy(table_h.at[scb], stage_b, sem_b)
                pltpu.make_async_copy(table_h.at[sca], stage_a,
                                      sem_a).wait()
                pltpu.sync_copy(stage_a, acc_s.at[dca], add=True)

                @pl.when(t < nch // 2 - 1)
                def _prefetch():
                    unpack(packed_v, j0 + 2, sca, dca)
                    pltpu.async_copy(table_h.at[sca], stage_a, sem_a)

                pltpu.make_async_copy(table_h.at[scb], stage_b,
                                      sem_b).wait()
                pltpu.sync_copy(stage_b, acc_s.at[dcb], add=True)
                return carry

            lax.fori_loop(0, nch // 2, body, 0)

        if t0 > 0:
            @pl.when(cid == 0)
            def _core0():
                for r0 in range(0, t0, tmax):
                    run(min(tmax, t0 - r0), sid * t0 + r0)

        if t1 > 0:
            @pl.when(cid == 1)
            def _core1():
                for r0 in range(0, t1, tmax):
                    run(min(tmax, t1 - r0), NS * t0 + sid * t1 + r0)

        plsc.subcore_barrier()
        pltpu.sync_copy(acc_s.at[pl.ds(row0, RPT)],
                        out_h.at[cid, pl.ds(row0, RPT)])

    return kern(table, packed, zeros)


def _deg_pass(dstp, ones, zeros):
    @functools.partial(
        pl.kernel,
        out_type=jax.ShapeDtypeStruct((NC, NP, 16), jnp.float32),
        mesh=_mesh(),
        compiler_params=pltpu.CompilerParams(use_tc_tiling_on_sc=False),
        scratch_types=[
            pltpu.VMEM((NCH, CK), jnp.int32),
            pltpu.VMEM((CK, 16), jnp.float32),
            pltpu.VMEM_SHARED((NP, 16), jnp.float32),
        ],
    )
    def kern(dst_h, ones_h, zeros_h, out_h, dst_v, stage_v, acc_s):
        cid = lax.axis_index("c")
        sid = lax.axis_index("s")
        wid = sid * NC + cid
        row0 = sid * RPT
        pltpu.sync_copy(ones_h, stage_v)
        pltpu.sync_copy(dst_h.at[pl.ds(wid * NCH, NCH)], dst_v)
        pltpu.sync_copy(zeros_h, acc_s.at[pl.ds(row0, RPT)])
        plsc.subcore_barrier()

        def chunk(j, carry):
            pltpu.sync_copy(stage_v, acc_s.at[dst_v.at[j]], add=True)
            return carry

        lax.fori_loop(0, NCH, chunk, 0)
        plsc.subcore_barrier()
        pltpu.sync_copy(acc_s.at[pl.ds(row0, RPT)],
                        out_h.at[cid, pl.ds(row0, RPT)])

    return kern(dstp, ones, zeros)


def _tc1(degp, x16):
    def body(degp_ref, x16_ref, dinv_ref, xs_ref):
        deg = degp_ref[0, 0:N, 0:1] + degp_ref[1, 0:N, 0:1] + 1.0
        dinv = lax.rsqrt(deg)
        dinv_ref[...] = dinv
        xs_ref[...] = dinv * x16_ref[...]

    return pl.pallas_call(
        body,
        out_shape=(
            jax.ShapeDtypeStruct((N, 1), jnp.float32),
            jax.ShapeDtypeStruct((N, 16), jnp.float32),
        ),
    )(degp, x16)


def _tc2(a1p, xs, dinv, w116, w2, b1):
    def body(a1p_ref, xs_ref, dinv_ref, w1_ref, w2_ref, b1_ref,
             h1_ref, g2_ref):
        dinv = dinv_ref[...]
        a = a1p_ref[0] + a1p_ref[1] + xs_ref[...]
        t = jnp.dot(a, w1_ref[...], preferred_element_type=jnp.float32)
        h1 = jnp.maximum(dinv * t + b1_ref[...], 0.0)
        h1_ref[...] = h1
        g2_ref[...] = dinv * jnp.dot(h1, w2_ref[...],
                                     preferred_element_type=jnp.float32)

    return pl.pallas_call(
        body,
        grid=(N // RB,),
        in_specs=[
            pl.BlockSpec((NC, RB, 16), lambda i: (0, i, 0)),
            pl.BlockSpec((RB, 16), lambda i: (i, 0)),
            pl.BlockSpec((RB, 1), lambda i: (i, 0)),
            pl.BlockSpec((16, HID), lambda i: (0, 0)),
            pl.BlockSpec((HID, HID), lambda i: (0, 0)),
            pl.BlockSpec((1, HID), lambda i: (0, 0)),
        ],
        out_specs=[
            pl.BlockSpec((RB, HID), lambda i: (i, 0)),
            pl.BlockSpec((RB, HID), lambda i: (i, 0)),
        ],
        out_shape=(
            jax.ShapeDtypeStruct((N, HID), jnp.float32),
            jax.ShapeDtypeStruct((N, HID), jnp.float32),
        ),
    )(a1p, xs, dinv, w116, w2, b1)


def _tc3(a2p, g2, dinv, b2):
    def body(a2p_ref, g2_ref, dinv_ref, b2_ref, out_ref):
        s = a2p_ref[0] + a2p_ref[1] + g2_ref[...]
        out_ref[...] = jnp.maximum(dinv_ref[...] * s + b2_ref[...], 0.0)

    return pl.pallas_call(
        body,
        grid=(N // RB,),
        in_specs=[
            pl.BlockSpec((NC, RB, HID), lambda i: (0, i, 0)),
            pl.BlockSpec((RB, HID), lambda i: (i, 0)),
            pl.BlockSpec((RB, 1), lambda i: (i, 0)),
            pl.BlockSpec((1, HID), lambda i: (0, 0)),
        ],
        out_specs=pl.BlockSpec((RB, HID), lambda i: (i, 0)),
        out_shape=jax.ShapeDtypeStruct((N, HID), jnp.float32),
    )(a2p, g2, dinv, b2)


def kernel(x, edge_index, W1, b1, W2, b2):
    src = edge_index[0].astype(jnp.int32)
    dst = edge_index[1].astype(jnp.int32)
    dstp = jnp.concatenate(
        [dst, jnp.full((EP - E,), N, jnp.int32)]).reshape(NW * NCH, CK)
    packed = jnp.concatenate(
        [src, jnp.zeros((EP - E,), jnp.int32)]).reshape(NW * NCH, CK) | (
            dstp << 16)
    x16 = jnp.pad(x, ((0, 0), (0, 16 - x.shape[1])))
    w116 = jnp.pad(W1, ((0, 16 - W1.shape[0]), (0, 0)))
    ones16 = jnp.ones((CK, 16), jnp.float32)
    zeros16 = jnp.zeros((RPT, 16), jnp.float32)
    zeros128 = jnp.zeros((RPT, HID), jnp.float32)

    degp = _deg_pass(dstp, ones16, zeros16)
    dinv, xs = _tc1(degp, x16)
    a1p = _seg_sum(xs, packed, zeros16, 16, (NCH, NCH))
    h1, g2 = _tc2(a1p, xs, dinv, w116, W2, b1.reshape(1, HID))
    a2p = _seg_sum(g2, packed, zeros128, HID, (128, 32))
    return _tc3(a2p, g2, dinv, b2.reshape(1, HID))


# speedup vs baseline: 1.3877x; 1.0642x over previous
import functools

import jax
import jax.numpy as jnp
from jax import lax
from jax.experimental import pallas as pl
from jax.experimental.pallas import tpu as pltpu
from jax.experimental.pallas import tpu_sc as plsc

N = 10000
E = 320000
HID = 128
NC = 2
NS = 16
NW = NC * NS
CK = 128
NCH = 80
EPT = CK * NCH
EP = EPT * NW
NP = 10016
RPT = NP // NS
RB = 1000

_mesh = functools.partial(
    plsc.VectorSubcoreMesh, core_axis_name="c", subcore_axis_name="s"
)


def _seg_sum(table, packed, zeros, d, split):
    def unpack(packed_v, j, scv, dcv):
        for v in range(CK // 16):
            p = packed_v[j, pl.ds(v * 16, 16)]
            scv[pl.ds(v * 16, 16)] = lax.bitwise_and(p, jnp.int32(0xFFFF))
            dcv[pl.ds(v * 16, 16)] = lax.shift_right_logical(p, 16)

    t0, t1 = split
    tmax = min(NCH, max(t0, t1))

    @functools.partial(
        pl.kernel,
        out_type=jax.ShapeDtypeStruct((NC, NP, d), jnp.float32),
        mesh=_mesh(),
        compiler_params=pltpu.CompilerParams(use_tc_tiling_on_sc=False),
        scratch_types=[
            pltpu.VMEM((tmax, CK), jnp.int32),
            pltpu.VMEM((CK,), jnp.int32),
            pltpu.VMEM((CK,), jnp.int32),
            pltpu.VMEM((CK,), jnp.int32),
            pltpu.VMEM((CK,), jnp.int32),
            pltpu.VMEM((CK, d), jnp.float32),
            pltpu.VMEM((CK, d), jnp.float32),
            pltpu.VMEM_SHARED((NP, d), jnp.float32),
            pltpu.SemaphoreType.DMA,
            pltpu.SemaphoreType.DMA,
        ],
    )
    def kern(table_h, packed_h, zeros_h, out_h, packed_v, sca, dca, scb,
             dcb, stage_a, stage_b, acc_s, sem_a, sem_b):
        cid = lax.axis_index("c")
        sid = lax.axis_index("s")
        row0 = sid * RPT
        pltpu.sync_copy(zeros_h, acc_s.at[pl.ds(row0, RPT)])

        def run(nch, gbase):
            pltpu.sync_copy(packed_h.at[pl.ds(gbase, nch)],
                            packed_v.at[pl.ds(0, nch)])
            plsc.subcore_barrier()
            unpack(packed_v, 0, sca, dca)
            pltpu.async_copy(table_h.at[sca], stage_a, sem_a)

            def body(t, carry):
                j0 = 2 * t
                unpack(packed_v, j0 + 1, scb, dcb)
                pltpu.async_copy(table_h.at[scb], stage_b, sem_b)
                pltpu.make_async_copy(table_h.at[sca], stage_a,
                                      sem_a).wait()
                pltpu.sync_copy(stage_a, acc_s.at[dca], add=True)

                @pl.when(t < nch // 2 - 1)
                def _prefetch():
                    unpack(packed_v, j0 + 2, sca, dca)
                    pltpu.async_copy(table_h.at[sca], stage_a, sem_a)

                pltpu.make_async_copy(table_h.at[scb], stage_b,
                                      sem_b).wait()
                pltpu.sync_copy(stage_b, acc_s.at[dcb], add=True)
                return carry

            lax.fori_loop(0, nch // 2, body, 0)

        if t0 > 0:
            @pl.when(cid == 0)
            def _core0():
                for r0 in range(0, t0, tmax):
                    run(min(tmax, t0 - r0), sid * t0 + r0)

        if t1 > 0:
            @pl.when(cid == 1)
            def _core1():
                for r0 in range(0, t1, tmax):
                    run(min(tmax, t1 - r0), NS * t0 + sid * t1 + r0)

        plsc.subcore_barrier()
        pltpu.sync_copy(acc_s.at[pl.ds(row0, RPT)],
                        out_h.at[cid, pl.ds(row0, RPT)])

    return kern(table, packed, zeros)


def _seg_sum_cols(ta, tb, packed, zeros):
    d = HID // 2
    TPC = NW * NCH // NS

    def unpack(packed_v, j, scv, dcv):
        for v in range(CK // 16):
            p = packed_v[j, pl.ds(v * 16, 16)]
            scv[pl.ds(v * 16, 16)] = lax.bitwise_and(p, jnp.int32(0xFFFF))
            dcv[pl.ds(v * 16, 16)] = lax.shift_right_logical(p, 16)

    @functools.partial(
        pl.kernel,
        out_type=jax.ShapeDtypeStruct((NC, NP, d), jnp.float32),
        mesh=_mesh(),
        compiler_params=pltpu.CompilerParams(use_tc_tiling_on_sc=False),
        scratch_types=[
            pltpu.VMEM((TPC, CK), jnp.int32),
            pltpu.VMEM((CK,), jnp.int32),
            pltpu.VMEM((CK,), jnp.int32),
            pltpu.VMEM((CK,), jnp.int32),
            pltpu.VMEM((CK,), jnp.int32),
            pltpu.VMEM((CK, d), jnp.float32),
            pltpu.VMEM((CK, d), jnp.float32),
            pltpu.VMEM_SHARED((NP, d), jnp.float32),
            pltpu.SemaphoreType.DMA,
            pltpu.SemaphoreType.DMA,
        ],
    )
    def kern(ta_h, tb_h, packed_h, zeros_h, out_h, packed_v, sca, dca, scb,
             dcb, stage_a, stage_b, acc_s, sem_a, sem_b):
        cid = lax.axis_index("c")
        sid = lax.axis_index("s")
        row0 = sid * RPT
        pltpu.sync_copy(zeros_h, acc_s.at[pl.ds(row0, RPT)])

        def run(table_h):
            pltpu.sync_copy(packed_h.at[pl.ds(sid * TPC, TPC)], packed_v)
            plsc.subcore_barrier()
            unpack(packed_v, 0, sca, dca)
            pltpu.async_copy(table_h.at[sca], stage_a, sem_a)

            def body(t, carry):
                j0 = 2 * t
                unpack(packed_v, j0 + 1, scb, dcb)
                pltpu.async_copy(table_h.at[scb], stage_b, sem_b)
                pltpu.make_async_copy(table_h.at[sca], stage_a,
                                      sem_a).wait()
                pltpu.sync_copy(stage_a, acc_s.at[dca], add=True)

                @pl.when(t < TPC // 2 - 1)
                def _prefetch():
                    unpack(packed_v, j0 + 2, sca, dca)
                    pltpu.async_copy(table_h.at[sca], stage_a, sem_a)

                pltpu.make_async_copy(table_h.at[scb], stage_b,
                                      sem_b).wait()
                pltpu.sync_copy(stage_b, acc_s.at[dcb], add=True)
                return carry

            lax.fori_loop(0, TPC // 2, body, 0)

        @pl.when(cid == 0)
        def _core0():
            run(ta_h)

        @pl.when(cid == 1)
        def _core1():
            run(tb_h)

        plsc.subcore_barrier()
        pltpu.sync_copy(acc_s.at[pl.ds(row0, RPT)],
                        out_h.at[cid, pl.ds(row0, RPT)])

    return kern(ta, tb, packed, zeros)


def _deg_pass(dstp, ones, zeros):
    @functools.partial(
        pl.kernel,
        out_type=jax.ShapeDtypeStruct((NC, NP, 16), jnp.float32),
        mesh=_mesh(),
        compiler_params=pltpu.CompilerParams(use_tc_tiling_on_sc=False),
        scratch_types=[
            pltpu.VMEM((NCH, CK), jnp.int32),
            pltpu.VMEM((CK, 16), jnp.float32),
            pltpu.VMEM_SHARED((NP, 16), jnp.float32),
        ],
    )
    def kern(dst_h, ones_h, zeros_h, out_h, dst_v, stage_v, acc_s):
        cid = lax.axis_index("c")
        sid = lax.axis_index("s")
        wid = sid * NC + cid
        row0 = sid * RPT
        pltpu.sync_copy(ones_h, stage_v)
        pltpu.sync_copy(dst_h.at[pl.ds(wid * NCH, NCH)], dst_v)
        pltpu.sync_copy(zeros_h, acc_s.at[pl.ds(row0, RPT)])
        plsc.subcore_barrier()

        def chunk(j, carry):
            pltpu.sync_copy(stage_v, acc_s.at[dst_v.at[j]], add=True)
            return carry

        lax.fori_loop(0, NCH, chunk, 0)
        plsc.subcore_barrier()
        pltpu.sync_copy(acc_s.at[pl.ds(row0, RPT)],
                        out_h.at[cid, pl.ds(row0, RPT)])

    return kern(dstp, ones, zeros)


def _tc1(degp, x16):
    def body(degp_ref, x16_ref, dinv_ref, xs_ref):
        deg = degp_ref[0, 0:N, 0:1] + degp_ref[1, 0:N, 0:1] + 1.0
        dinv = lax.rsqrt(deg)
        dinv_ref[...] = dinv
        xs_ref[...] = dinv * x16_ref[...]

    return pl.pallas_call(
        body,
        out_shape=(
            jax.ShapeDtypeStruct((N, 1), jnp.float32),
            jax.ShapeDtypeStruct((N, 16), jnp.float32),
        ),
    )(degp, x16)


def _tc2(a1p, xs, dinv, w116, w2, b1):
    def body(a1p_ref, xs_ref, dinv_ref, w1_ref, w2_ref, b1_ref,
             h1_ref, g2_ref):
        dinv = dinv_ref[...]
        a = a1p_ref[0] + a1p_ref[1] + xs_ref[...]
        t = jnp.dot(a, w1_ref[...], preferred_element_type=jnp.float32)
        h1 = jnp.maximum(dinv * t + b1_ref[...], 0.0)
        h1_ref[...] = h1
        g2_ref[...] = dinv * jnp.dot(h1, w2_ref[...],
                                     preferred_element_type=jnp.float32)

    return pl.pallas_call(
        body,
        grid=(N // RB,),
        in_specs=[
            pl.BlockSpec((NC, RB, 16), lambda i: (0, i, 0)),
            pl.BlockSpec((RB, 16), lambda i: (i, 0)),
            pl.BlockSpec((RB, 1), lambda i: (i, 0)),
            pl.BlockSpec((16, HID), lambda i: (0, 0)),
            pl.BlockSpec((HID, HID), lambda i: (0, 0)),
            pl.BlockSpec((1, HID), lambda i: (0, 0)),
        ],
        out_specs=[
            pl.BlockSpec((RB, HID), lambda i: (i, 0)),
            pl.BlockSpec((RB, HID), lambda i: (i, 0)),
        ],
        out_shape=(
            jax.ShapeDtypeStruct((N, HID), jnp.float32),
            jax.ShapeDtypeStruct((N, HID), jnp.float32),
        ),
    )(a1p, xs, dinv, w116, w2, b1)


def _tc3(a2p, g2, dinv, b2):
    def body(a2p_ref, g2_ref, dinv_ref, b2_ref, out_ref):
        s = jnp.concatenate([a2p_ref[0], a2p_ref[1]], axis=1) + g2_ref[...]
        out_ref[...] = jnp.maximum(dinv_ref[...] * s + b2_ref[...], 0.0)

    return pl.pallas_call(
        body,
        grid=(N // RB,),
        in_specs=[
            pl.BlockSpec((NC, RB, HID // 2), lambda i: (0, i, 0)),
            pl.BlockSpec((RB, HID), lambda i: (i, 0)),
            pl.BlockSpec((RB, 1), lambda i: (i, 0)),
            pl.BlockSpec((1, HID), lambda i: (0, 0)),
        ],
        out_specs=pl.BlockSpec((RB, HID), lambda i: (i, 0)),
        out_shape=jax.ShapeDtypeStruct((N, HID), jnp.float32),
    )(a2p, g2, dinv, b2)


def kernel(x, edge_index, W1, b1, W2, b2):
    src = edge_index[0].astype(jnp.int32)
    dst = edge_index[1].astype(jnp.int32)
    dstp = jnp.concatenate(
        [dst, jnp.full((EP - E,), N, jnp.int32)]).reshape(NW * NCH, CK)
    packed = jnp.concatenate(
        [src, jnp.zeros((EP - E,), jnp.int32)]).reshape(NW * NCH, CK) | (
            dstp << 16)
    x16 = jnp.pad(x, ((0, 0), (0, 16 - x.shape[1])))
    w116 = jnp.pad(W1, ((0, 16 - W1.shape[0]), (0, 0)))
    ones16 = jnp.ones((CK, 16), jnp.float32)
    zeros16 = jnp.zeros((RPT, 16), jnp.float32)
    zeros64 = jnp.zeros((RPT, HID // 2), jnp.float32)

    degp = _deg_pass(dstp, ones16, zeros16)
    dinv, xs = _tc1(degp, x16)
    a1p = _seg_sum(xs, packed, zeros16, 16, (NCH, NCH))
    h1, g2 = _tc2(a1p, xs, dinv, w116, W2, b1.reshape(1, HID))
    a2p = _seg_sum_cols(g2[:, :HID // 2], g2[:, HID // 2:], packed,
                        zeros64)
    return _tc3(a2p, g2, dinv, b2.reshape(1, HID))


# speedup vs baseline: 1.4336x; 1.0331x over previous
import functools

import jax
import jax.numpy as jnp
from jax import lax
from jax.experimental import pallas as pl
from jax.experimental.pallas import tpu as pltpu
from jax.experimental.pallas import tpu_sc as plsc

N = 10000
E = 320000
HID = 128
NC = 2
NS = 16
NW = NC * NS
CK = 128
NCH = 80
EPT = CK * NCH
EP = EPT * NW
NP = 10016
RPT = NP // NS
RB = 1000

_mesh = functools.partial(
    plsc.VectorSubcoreMesh, core_axis_name="c", subcore_axis_name="s"
)


def _seg_sum(table, packed, zeros, d, split):
    def unpack(packed_v, j, scv, dcv):
        for v in range(CK // 16):
            p = packed_v[j, pl.ds(v * 16, 16)]
            scv[pl.ds(v * 16, 16)] = lax.bitwise_and(p, jnp.int32(0xFFFF))
            dcv[pl.ds(v * 16, 16)] = lax.shift_right_logical(p, 16)

    t0, t1 = split
    tmax = min(NCH, max(t0, t1))

    @functools.partial(
        pl.kernel,
        out_type=jax.ShapeDtypeStruct((NC, NP, d), jnp.float32),
        mesh=_mesh(),
        compiler_params=pltpu.CompilerParams(use_tc_tiling_on_sc=False),
        scratch_types=[
            pltpu.VMEM((tmax, CK), jnp.int32),
            pltpu.VMEM((CK,), jnp.int32),
            pltpu.VMEM((CK,), jnp.int32),
            pltpu.VMEM((CK,), jnp.int32),
            pltpu.VMEM((CK,), jnp.int32),
            pltpu.VMEM((CK, d), jnp.float32),
            pltpu.VMEM((CK, d), jnp.float32),
            pltpu.VMEM_SHARED((NP, d), jnp.float32),
            pltpu.SemaphoreType.DMA,
            pltpu.SemaphoreType.DMA,
        ],
    )
    def kern(table_h, packed_h, zeros_h, out_h, packed_v, sca, dca, scb,
             dcb, stage_a, stage_b, acc_s, sem_a, sem_b):
        cid = lax.axis_index("c")
        sid = lax.axis_index("s")
        row0 = sid * RPT
        pltpu.sync_copy(zeros_h, acc_s.at[pl.ds(row0, RPT)])

        def run(nch, gbase):
            pltpu.sync_copy(packed_h.at[pl.ds(gbase, nch)],
                            packed_v.at[pl.ds(0, nch)])
            plsc.subcore_barrier()
            unpack(packed_v, 0, sca, dca)
            pltpu.async_copy(table_h.at[sca], stage_a, sem_a)

            def body(t, carry):
                j0 = 2 * t
                unpack(packed_v, j0 + 1, scb, dcb)
                pltpu.async_copy(table_h.at[scb], stage_b, sem_b)
                pltpu.make_async_copy(table_h.at[sca], stage_a,
                                      sem_a).wait()
                pltpu.sync_copy(stage_a, acc_s.at[dca], add=True)

                @pl.when(t < nch // 2 - 1)
                def _prefetch():
                    unpack(packed_v, j0 + 2, sca, dca)
                    pltpu.async_copy(table_h.at[sca], stage_a, sem_a)

                pltpu.make_async_copy(table_h.at[scb], stage_b,
                                      sem_b).wait()
                pltpu.sync_copy(stage_b, acc_s.at[dcb], add=True)
                return carry

            lax.fori_loop(0, nch // 2, body, 0)

        if t0 > 0:
            @pl.when(cid == 0)
            def _core0():
                for r0 in range(0, t0, tmax):
                    run(min(tmax, t0 - r0), sid * t0 + r0)

        if t1 > 0:
            @pl.when(cid == 1)
            def _core1():
                for r0 in range(0, t1, tmax):
                    run(min(tmax, t1 - r0), NS * t0 + sid * t1 + r0)

        plsc.subcore_barrier()
        pltpu.sync_copy(acc_s.at[pl.ds(row0, RPT)],
                        out_h.at[cid, pl.ds(row0, RPT)])

    return kern(table, packed, zeros)


def _seg_sum_cols(ta, tb, packed, zeros):
    d = HID // 2
    TPC = NW * NCH // NS

    def unpack(packed_v, j, scv, dcv):
        for v in range(CK // 16):
            p = packed_v[j, pl.ds(v * 16, 16)]
            scv[pl.ds(v * 16, 16)] = lax.bitwise_and(p, jnp.int32(0xFFFF))
            dcv[pl.ds(v * 16, 16)] = lax.shift_right_logical(p, 16)

    @functools.partial(
        pl.kernel,
        out_type=jax.ShapeDtypeStruct((NC, NP, d), jnp.float32),
        mesh=_mesh(),
        compiler_params=pltpu.CompilerParams(use_tc_tiling_on_sc=False),
        scratch_types=[
            pltpu.VMEM((TPC, CK), jnp.int32),
            pltpu.VMEM((CK,), jnp.int32),
            pltpu.VMEM((CK,), jnp.int32),
            pltpu.VMEM((CK,), jnp.int32),
            pltpu.VMEM((CK,), jnp.int32),
            pltpu.VMEM((CK, d), jnp.float32),
            pltpu.VMEM((CK, d), jnp.float32),
            pltpu.VMEM_SHARED((NP, d), jnp.float32),
            pltpu.SemaphoreType.DMA,
            pltpu.SemaphoreType.DMA,
        ],
    )
    def kern(ta_h, tb_h, packed_h, zeros_h, out_h, packed_v, sca, dca, scb,
             dcb, stage_a, stage_b, acc_s, sem_a, sem_b):
        cid = lax.axis_index("c")
        sid = lax.axis_index("s")
        row0 = sid * RPT
        pltpu.sync_copy(zeros_h, acc_s.at[pl.ds(row0, RPT)])

        def run(table_h):
            pltpu.sync_copy(packed_h.at[pl.ds(sid * TPC, TPC)], packed_v)
            plsc.subcore_barrier()
            unpack(packed_v, 0, sca, dca)
            pltpu.async_copy(table_h.at[sca], stage_a, sem_a)

            def body(t, carry):
                j0 = 2 * t
                unpack(packed_v, j0 + 1, scb, dcb)
                pltpu.async_copy(table_h.at[scb], stage_b, sem_b)
                pltpu.make_async_copy(table_h.at[sca], stage_a,
                                      sem_a).wait()
                pltpu.sync_copy(stage_a, acc_s.at[dca], add=True)

                @pl.when(t < TPC // 2 - 1)
                def _prefetch():
                    unpack(packed_v, j0 + 2, sca, dca)
                    pltpu.async_copy(table_h.at[sca], stage_a, sem_a)

                pltpu.make_async_copy(table_h.at[scb], stage_b,
                                      sem_b).wait()
                pltpu.sync_copy(stage_b, acc_s.at[dcb], add=True)
                return carry

            lax.fori_loop(0, TPC // 2, body, 0)

        @pl.when(cid == 0)
        def _core0():
            run(ta_h)

        @pl.when(cid == 1)
        def _core1():
            run(tb_h)

        plsc.subcore_barrier()
        pltpu.sync_copy(acc_s.at[pl.ds(row0, RPT)],
                        out_h.at[cid, pl.ds(row0, RPT)])

    return kern(ta, tb, packed, zeros)


def _deg_pass(dstp, ones, zeros):
    @functools.partial(
        pl.kernel,
        out_type=jax.ShapeDtypeStruct((NC, NP, 16), jnp.float32),
        mesh=_mesh(),
        compiler_params=pltpu.CompilerParams(use_tc_tiling_on_sc=False),
        scratch_types=[
            pltpu.VMEM((NCH, CK), jnp.int32),
            pltpu.VMEM((CK, 16), jnp.float32),
            pltpu.VMEM_SHARED((NP, 16), jnp.float32),
        ],
    )
    def kern(dst_h, ones_h, zeros_h, out_h, dst_v, stage_v, acc_s):
        cid = lax.axis_index("c")
        sid = lax.axis_index("s")
        wid = sid * NC + cid
        row0 = sid * RPT
        pltpu.sync_copy(ones_h, stage_v)
        pltpu.sync_copy(dst_h.at[pl.ds(wid * NCH, NCH)], dst_v)
        pltpu.sync_copy(zeros_h, acc_s.at[pl.ds(row0, RPT)])
        plsc.subcore_barrier()

        def chunk(j, carry):
            pltpu.sync_copy(stage_v, acc_s.at[dst_v.at[j]], add=True)
            return carry

        lax.fori_loop(0, NCH, chunk, 0)
        plsc.subcore_barrier()
        pltpu.sync_copy(acc_s.at[pl.ds(row0, RPT)],
                        out_h.at[cid, pl.ds(row0, RPT)])

    return kern(dstp, ones, zeros)


def _tc1(degp, x16):
    def body(degp_ref, x16_ref, dinv_ref, xs_ref):
        deg = degp_ref[0, 0:N, 0:1] + degp_ref[1, 0:N, 0:1] + 1.0
        dinv = lax.rsqrt(deg)
        dinv_ref[...] = dinv
        xs_ref[...] = dinv * x16_ref[...]

    return pl.pallas_call(
        body,
        out_shape=(
            jax.ShapeDtypeStruct((N, 1), jnp.float32),
            jax.ShapeDtypeStruct((N, 16), jnp.float32),
        ),
    )(degp, x16)


def _tc2(a1p, xs, dinv, w116, w2, b1):
    def body(a1p_ref, xs_ref, dinv_ref, w1_ref, w2_ref, b1_ref,
             h1_ref, g2_ref):
        dinv = dinv_ref[...]
        a = a1p_ref[0] + a1p_ref[1] + xs_ref[...]
        t = jnp.dot(a, w1_ref[...], preferred_element_type=jnp.float32)
        h1 = jnp.maximum(dinv * t + b1_ref[...], 0.0)
        h1_ref[...] = h1
        g2_ref[...] = dinv * jnp.dot(h1, w2_ref[...],
                                     preferred_element_type=jnp.float32)

    return pl.pallas_call(
        body,
        grid=(N // RB,),
        in_specs=[
            pl.BlockSpec((NC, RB, 16), lambda i: (0, i, 0)),
            pl.BlockSpec((RB, 16), lambda i: (i, 0)),
            pl.BlockSpec((RB, 1), lambda i: (i, 0)),
            pl.BlockSpec((16, HID), lambda i: (0, 0)),
            pl.BlockSpec((HID, HID), lambda i: (0, 0)),
            pl.BlockSpec((1, HID), lambda i: (0, 0)),
        ],
        out_specs=[
            pl.BlockSpec((RB, HID), lambda i: (i, 0)),
            pl.BlockSpec((RB, HID), lambda i: (i, 0)),
        ],
        out_shape=(
            jax.ShapeDtypeStruct((N, HID), jnp.float32),
            jax.ShapeDtypeStruct((N, HID), jnp.float32),
        ),
    )(a1p, xs, dinv, w116, w2, b1)


def _tc3(a2p, g2, dinv, b2):
    def body(a2p_ref, g2_ref, dinv_ref, b2_ref, out_ref):
        s = jnp.concatenate([a2p_ref[0], a2p_ref[1]], axis=1) + g2_ref[...]
        out_ref[...] = jnp.maximum(dinv_ref[...] * s + b2_ref[...], 0.0)

    return pl.pallas_call(
        body,
        grid=(N // RB,),
        in_specs=[
            pl.BlockSpec((NC, RB, HID // 2), lambda i: (0, i, 0)),
            pl.BlockSpec((RB, HID), lambda i: (i, 0)),
            pl.BlockSpec((RB, 1), lambda i: (i, 0)),
            pl.BlockSpec((1, HID), lambda i: (0, 0)),
        ],
        out_specs=pl.BlockSpec((RB, HID), lambda i: (i, 0)),
        out_shape=jax.ShapeDtypeStruct((N, HID), jnp.float32),
    )(a2p, g2, dinv, b2)


def kernel(x, edge_index, W1, b1, W2, b2):
    src = edge_index[0].astype(jnp.int32)
    dst = edge_index[1].astype(jnp.int32)
    dstp = jnp.concatenate(
        [dst, jnp.full((EP - E,), N, jnp.int32)]).reshape(NW * NCH, CK)
    packed = jnp.concatenate(
        [src, jnp.zeros((EP - E,), jnp.int32)]).reshape(NW * NCH, CK) | (
            dstp << 16)
    x16 = jnp.pad(x, ((0, 0), (0, 16 - x.shape[1])))
    w116 = jnp.pad(W1, ((0, 16 - W1.shape[0]), (0, 0)))
    ones16 = jnp.ones((CK, 16), jnp.float32)
    zeros16 = jnp.zeros((RPT, 16), jnp.float32)
    zeros64 = jnp.zeros((RPT, HID // 2), jnp.float32)

    degp = _deg_pass(dstp, ones16, zeros16)
    dinv, xs = _tc1(degp, x16)
    a1p = _seg_sum(xs, packed, zeros16, 16, (96, 64))
    h1, g2 = _tc2(a1p, xs, dinv, w116, W2, b1.reshape(1, HID))
    a2p = _seg_sum_cols(g2[:, :HID // 2], g2[:, HID // 2:], packed,
                        zeros64)
    return _tc3(a2p, g2, dinv, b2.reshape(1, HID))
